# baseline (speedup 1.0000x reference)
import functools

import jax
import jax.numpy as jnp
from jax import lax
from jax.experimental import pallas as pl
from jax.experimental.pallas import tpu as pltpu
from jax.experimental.pallas import tpu_sc as plsc

B = 16384
VOCAB = 1000
EMB = 16
ND = 13
H = 16
NIN = ND + 2 * EMB

NC = 2
NS = 16
NW = NC * NS
BPW = B // NW
L = 16
NCHK = BPW // L

_mesh = plsc.VectorSubcoreMesh(core_axis_name="c", subcore_axis_name="s")


@functools.partial(
    pl.kernel,
    mesh=_mesh,
    compiler_params=pltpu.CompilerParams(use_tc_tiling_on_sc=False,
                                         needs_layout_passes=False),
    out_type=jax.ShapeDtypeStruct((B,), jnp.float32),
    scratch_types=[
        pltpu.VMEM((BPW * ND,), jnp.float32),
        pltpu.VMEM((BPW,), jnp.int32),
        pltpu.VMEM((BPW,), jnp.int32),
        pltpu.VMEM((VOCAB * EMB,), jnp.float32),
        pltpu.VMEM((VOCAB * EMB,), jnp.float32),
        pltpu.VMEM((NIN, H), jnp.float32),
        pltpu.VMEM((H,), jnp.float32),
        pltpu.VMEM((H,), jnp.float32),
        pltpu.VMEM((1,), jnp.float32),
        pltpu.SMEM((NIN, H), jnp.float32),
        pltpu.SMEM((H,), jnp.float32),
        pltpu.SMEM((H,), jnp.float32),
        pltpu.VMEM((BPW,), jnp.float32),
        pltpu.SemaphoreType.DMA,
    ],
)
def _sc_fused(i1_hbm, c1_hbm, c2_hbm, emb1_hbm, emb2_hbm, w1_hbm, b1_hbm,
              w2_hbm, b2_hbm, out_hbm,
              i1_v, c1_v, c2_v, e1_v, e2_v, w1_vm, b1_vm, w2_vm, b2_vm,
              w1_s, b1_s, w2_s, out_v, sem):
    wid = lax.axis_index("s") * NC + lax.axis_index("c")
    base = wid * BPW
    cps = [
        pltpu.async_copy(i1_hbm.at[pl.ds(base * ND, BPW * ND)], i1_v, sem),
        pltpu.async_copy(c1_hbm.at[pl.ds(base, BPW)], c1_v, sem),
        pltpu.async_copy(c2_hbm.at[pl.ds(base, BPW)], c2_v, sem),
        pltpu.async_copy(emb1_hbm, e1_v, sem),
        pltpu.async_copy(emb2_hbm, e2_v, sem),
        pltpu.async_copy(w1_hbm, w1_vm, sem),
        pltpu.async_copy(b1_hbm, b1_vm, sem),
        pltpu.async_copy(w2_hbm, w2_vm, sem),
        pltpu.async_copy(b2_hbm, b2_vm, sem),
    ]
    for cp in cps:
        cp.wait()

    for k in range(NIN):
        row = w1_vm[k]
        for j in range(H):
            w1_s[k, j] = row[j]
    b1row = b1_vm[...]
    w2row = w2_vm[...]
    for j in range(H):
        b1_s[j] = b1row[j]
        w2_s[j] = w2row[j]

    lanes = lax.iota(jnp.int32, L)
    zeros = jnp.zeros((L,), jnp.int32)
    b2vec = plsc.load_gather(b2_vm, [zeros])

    def chunk(c, _):
        row0 = c * L
        rows13 = (lanes + row0) * ND
        c1v = c1_v[pl.ds(row0, L)] * EMB
        c2v = c2_v[pl.ds(row0, L)] * EMB
        acc = [jnp.full((L,), b1_s[j], jnp.float32) for j in range(H)]
        for k in range(ND):
            col = plsc.load_gather(i1_v, [rows13 + k])
            for j in range(H):
                acc[j] = acc[j] + col * w1_s[k, j]
        for jp in range(EMB):
            col = plsc.load_gather(e1_v, [c1v + jp])
            for j in range(H):
                acc[j] = acc[j] + col * w1_s[ND + jp, j]
        for jp in range(EMB):
            col = plsc.load_gather(e2_v, [c2v + jp])
            for j in range(H):
                acc[j] = acc[j] + col * w1_s[ND + EMB + jp, j]
        out = b2vec
        for j in range(H):
            out = out + jnp.maximum(acc[j], 0.0) * w2_s[j]
        out_v[pl.ds(row0, L)] = out
        return ()

    lax.fori_loop(0, NCHK, chunk, (), unroll=False)
    pltpu.sync_copy(out_v, out_hbm.at[pl.ds(base, BPW)])


def kernel(I1, C1, C2, emb1, emb2, W1, b1, W2, b2):
    out = _sc_fused(
        I1.reshape(B * ND),
        C1.astype(jnp.int32).reshape(B),
        C2.astype(jnp.int32).reshape(B),
        emb1.reshape(VOCAB * EMB),
        emb2.reshape(VOCAB * EMB),
        W1, b1, W2.reshape(H), b2)
    return out.reshape(B, 1)


# speedup vs baseline: 2.2142x; 2.2142x over previous
import functools

import jax
import jax.numpy as jnp
from jax import lax
from jax.experimental import pallas as pl
from jax.experimental.pallas import tpu as pltpu
from jax.experimental.pallas import tpu_sc as plsc

B = 16384
VOCAB = 1000
EMB = 16
ND = 13
H = 16
NIN = ND + 2 * EMB

NC = 2
NS = 16
NW = NC * NS
BPW = B // NW
L = 16
RPC = 2 * L
NCHK = BPW // RPC

W2_OFF = NIN * H
B1_OFF = W2_OFF + H
NSPLAT = B1_OFF + H

_mesh = plsc.VectorSubcoreMesh(core_axis_name="c", subcore_axis_name="s")


@functools.partial(
    pl.kernel,
    mesh=_mesh,
    compiler_params=pltpu.CompilerParams(use_tc_tiling_on_sc=False,
                                         needs_layout_passes=False),
    out_type=jax.ShapeDtypeStruct((B,), jnp.float32),
    scratch_types=[
        pltpu.VMEM((BPW * ND,), jnp.float32),
        pltpu.VMEM((BPW,), jnp.int32),
        pltpu.VMEM((BPW,), jnp.int32),
        pltpu.VMEM((VOCAB * EMB,), jnp.float32),
        pltpu.VMEM((VOCAB * EMB,), jnp.float32),
        pltpu.VMEM((NIN, H), jnp.float32),
        pltpu.VMEM((H,), jnp.float32),
        pltpu.VMEM((H,), jnp.float32),
        pltpu.VMEM((1,), jnp.float32),
        pltpu.VMEM((NSPLAT, L), jnp.float32),
        pltpu.VMEM((BPW,), jnp.float32),
        pltpu.SemaphoreType.DMA,
    ],
)
def _sc_fused(i1_hbm, c1_hbm, c2_hbm, emb1_hbm, emb2_hbm, w1_hbm, b1_hbm,
              w2_hbm, b2_hbm, out_hbm,
              i1_v, c1_v, c2_v, e1_v, e2_v, w1_vm, b1_vm, w2_vm, b2_vm,
              wsp, out_v, sem):
    wid = lax.axis_index("s") * NC + lax.axis_index("c")
    base = wid * BPW
    cps = [
        pltpu.async_copy(i1_hbm.at[pl.ds(base * ND, BPW * ND)], i1_v, sem),
        pltpu.async_copy(c1_hbm.at[pl.ds(base, BPW)], c1_v, sem),
        pltpu.async_copy(c2_hbm.at[pl.ds(base, BPW)], c2_v, sem),
        pltpu.async_copy(emb1_hbm, e1_v, sem),
        pltpu.async_copy(emb2_hbm, e2_v, sem),
        pltpu.async_copy(w1_hbm, w1_vm, sem),
        pltpu.async_copy(b1_hbm, b1_vm, sem),
        pltpu.async_copy(w2_hbm, w2_vm, sem),
        pltpu.async_copy(b2_hbm, b2_vm, sem),
    ]
    for cp in cps:
        cp.wait()

    for k in range(NIN):
        row = w1_vm[k]
        for j in range(H):
            wsp[k * H + j] = jnp.full((L,), row[j], jnp.float32)
    w2row = w2_vm[...]
    b1row = b1_vm[...]
    for j in range(H):
        wsp[W2_OFF + j] = jnp.full((L,), w2row[j], jnp.float32)
        wsp[B1_OFF + j] = jnp.full((L,), b1row[j], jnp.float32)

    lanes = lax.iota(jnp.int32, L)
    zeros = jnp.zeros((L,), jnp.int32)
    b2vec = plsc.load_gather(b2_vm, [zeros])

    def chunk(c, _):
        row0 = c * RPC
        rows_a = lanes + row0
        rows_b = rows_a + L
        rb13a = rows_a * ND
        rb13b = rows_b * ND
        c1a = c1_v[pl.ds(row0, L)] * EMB
        c1b = c1_v[pl.ds(row0 + L, L)] * EMB
        c2a = c2_v[pl.ds(row0, L)] * EMB
        c2b = c2_v[pl.ds(row0 + L, L)] * EMB
        binit = [wsp[B1_OFF + j] for j in range(H)]
        acca = list(binit)
        accb = list(binit)

        def fma_block(cola, colb, woff, acca, accb):
            for j in range(H):
                w = wsp[woff + j]
                acca[j] = acca[j] + cola * w
                accb[j] = accb[j] + colb * w

        for k in range(ND):
            fma_block(plsc.load_gather(i1_v, [rb13a + k]),
                      plsc.load_gather(i1_v, [rb13b + k]), k * H, acca, accb)
        for jp in range(EMB):
            fma_block(plsc.load_gather(e1_v, [c1a + jp]),
                      plsc.load_gather(e1_v, [c1b + jp]),
                      (ND + jp) * H, acca, accb)
        for jp in range(EMB):
            fma_block(plsc.load_gather(e2_v, [c2a + jp]),
                      plsc.load_gather(e2_v, [c2b + jp]),
                      (ND + EMB + jp) * H, acca, accb)
        outa = b2vec
        outb = b2vec
        for j in range(H):
            w2j = wsp[W2_OFF + j]
            outa = outa + jnp.maximum(acca[j], 0.0) * w2j
            outb = outb + jnp.maximum(accb[j], 0.0) * w2j
        out_v[pl.ds(row0, L)] = outa
        out_v[pl.ds(row0 + L, L)] = outb
        return ()

    lax.fori_loop(0, NCHK, chunk, (), unroll=False)
    pltpu.sync_copy(out_v, out_hbm.at[pl.ds(base, BPW)])


def kernel(I1, C1, C2, emb1, emb2, W1, b1, W2, b2):
    out = _sc_fused(
        I1.reshape(B * ND),
        C1.astype(jnp.int32).reshape(B),
        C2.astype(jnp.int32).reshape(B),
        emb1.reshape(VOCAB * EMB),
        emb2.reshape(VOCAB * EMB),
        W1, b1, W2.reshape(H), b2)
    return out.reshape(B, 1)
